# baseline (speedup 1.0000x reference)
import functools

import jax
import jax.numpy as jnp
from jax import lax
from jax.experimental import pallas as pl
from jax.experimental.pallas import tpu as pltpu
from jax.experimental.pallas import tpu_sc as plsc

N = 10000
E = 100000
IN = 128
OUT = 128
R = 3
NC = 2
NT = 16
NW = NC * NT
GROUP = 128
NBUF = 2
CH = 4
NCHUNK = 19
NGRP = CH * NCHUNK
EPAD = NW * NGRP * GROUP
ROWS_PAD = 10112
ROWS_PER_TILE = ROWS_PAD // NT

_BM = 1000


def _tc_matmul_body(x_ref, w_ref, y_ref):
    y_ref[...] = jnp.dot(x_ref[...], w_ref[0], preferred_element_type=jnp.float32)


def _tc_matmul(x, weight):
    return pl.pallas_call(
        _tc_matmul_body,
        grid=(R, N // _BM),
        in_specs=[
            pl.BlockSpec((_BM, IN), lambda r, i: (i, 0)),
            pl.BlockSpec((1, IN, OUT), lambda r, i: (r, 0, 0)),
        ],
        out_specs=pl.BlockSpec((_BM, OUT), lambda r, i: (r * (N // _BM) + i, 0)),
        out_shape=jax.ShapeDtypeStruct((R * N, OUT), jnp.float32),
    )(x, weight)


def _sc_scatter_body(ytab_hbm, isrc_hbm, idst_hbm, out_hbm,
                     s0_v, s1_v, d0_v, d1_v, b0, b1, agg_sp,
                     g0, g1, si0, si1, di0, di1):
    c = lax.axis_index("c")
    s = lax.axis_index("s")
    w = c * NT + s
    sbuf = (s0_v, s1_v)
    dbuf = (d0_v, d1_v)
    bufs = (b0, b1)
    gsem = (g0, g1)
    isem = (si0, si1)
    dsem = (di0, di1)

    def load_idx_chunk(k, ring, wait):
        src_hbm = isrc_hbm.at[w, k]
        dst_hbm = idst_hbm.at[w, k]
        if wait:
            pltpu.sync_copy(src_hbm, sbuf[ring])
            pltpu.sync_copy(dst_hbm, dbuf[ring])
        else:
            pltpu.async_copy(src_hbm, sbuf[ring], isem[ring])
            pltpu.async_copy(dst_hbm, dbuf[ring], dsem[ring])

    def wait_idx_chunk(ring):
        pltpu.make_async_copy(isrc_hbm.at[w, 0], sbuf[ring], isem[ring]).wait()
        pltpu.make_async_copy(idst_hbm.at[w, 0], dbuf[ring], dsem[ring]).wait()

    def fire_gather(b, ring, grp):
        pltpu.async_copy(ytab_hbm.at[sbuf[ring].at[grp]], bufs[b], gsem[b])

    def wait_gather(b):
        pltpu.make_async_copy(ytab_hbm.at[sbuf[0].at[0]], bufs[b], gsem[b]).wait()

    zval = jnp.zeros((16,), jnp.float32)

    def zero_row(i, carry):
        for j in range(OUT // 16):
            b0[i, pl.ds(j * 16, 16)] = zval
        return carry

    lax.fori_loop(0, GROUP, zero_row, 0)
    row0 = s * ROWS_PER_TILE
    for k in range(ROWS_PER_TILE // GROUP):
        pltpu.sync_copy(b0, agg_sp.at[pl.ds(row0 + k * GROUP, GROUP)])
    rem = ROWS_PER_TILE % GROUP
    if rem:
        pltpu.sync_copy(b0.at[pl.ds(0, rem)],
                        agg_sp.at[pl.ds(row0 + (ROWS_PER_TILE // GROUP) * GROUP, rem)])
    plsc.subcore_barrier()

    load_idx_chunk(0, 0, wait=True)
    for b in range(NBUF):
        fire_gather(b, 0, b)
    load_idx_chunk(1, 1, wait=False)

    def body(k, carry):
        p = lax.rem(k, 2)
        q = lax.rem(k + 1, 2)

        def on_ring(ring_p, ring_q):
            @pl.when(k + 1 < NCHUNK)
            def _():
                wait_idx_chunk(ring_q)
            for b in range(NBUF):
                wait_gather(b)
                pltpu.sync_copy(bufs[b], agg_sp.at[dbuf[ring_p].at[b]], add=True)
                fire_gather(b, ring_p, NBUF + b)
            for b in range(NBUF):
                wait_gather(b)
                pltpu.sync_copy(bufs[b], agg_sp.at[dbuf[ring_p].at[NBUF + b]], add=True)

                @pl.when(k + 1 < NCHUNK)
                def _():
                    fire_gather(b, ring_q, b)

            @pl.when(k + 2 < NCHUNK)
            def _():
                load_idx_chunk(k + 2, ring_p, wait=False)

        @pl.when(p == 0)
        def _():
            on_ring(0, 1)

        @pl.when(p == 1)
        def _():
            on_ring(1, 0)
        return carry

    lax.fori_loop(0, NCHUNK, body, 0)
    plsc.subcore_barrier()
    pltpu.sync_copy(agg_sp.at[pl.ds(row0, ROWS_PER_TILE)],
                    out_hbm.at[c, pl.ds(row0, ROWS_PER_TILE)])


_sc_scatter = functools.partial(
    pl.kernel,
    out_type=jax.ShapeDtypeStruct((NC, ROWS_PAD, OUT), jnp.float32),
    mesh=plsc.VectorSubcoreMesh(core_axis_name="c", subcore_axis_name="s"),
    scratch_types=[
        pltpu.VMEM((CH, GROUP), jnp.int32),
        pltpu.VMEM((CH, GROUP), jnp.int32),
        pltpu.VMEM((CH, GROUP), jnp.int32),
        pltpu.VMEM((CH, GROUP), jnp.int32),
        pltpu.VMEM((GROUP, OUT), jnp.float32),
        pltpu.VMEM((GROUP, OUT), jnp.float32),
        pltpu.VMEM_SHARED((ROWS_PAD, OUT), jnp.float32),
        pltpu.SemaphoreType.DMA,
        pltpu.SemaphoreType.DMA,
        pltpu.SemaphoreType.DMA,
        pltpu.SemaphoreType.DMA,
        pltpu.SemaphoreType.DMA,
        pltpu.SemaphoreType.DMA,
    ],
)(_sc_scatter_body)


def kernel(x, edge_index_r0, edge_index_r1, edge_index_r2, weight, h_bias):
    ytab = _tc_matmul(x, weight)

    gidx = jnp.concatenate([
        edge_index_r0[0],
        edge_index_r1[0] + N,
        edge_index_r2[0] + 2 * N,
    ])
    dst = jnp.concatenate([edge_index_r0[1], edge_index_r1[1], edge_index_r2[1]])
    pad = EPAD - R * E
    pad_iota = jax.lax.iota(jnp.int32, pad)
    gidx = jnp.concatenate([gidx, pad_iota % (R * N)])
    dst = jnp.concatenate([dst, N + pad_iota % (ROWS_PAD - N)])
    isrc = gidx.reshape(NW, NCHUNK, CH, GROUP)
    idst = dst.reshape(NW, NCHUNK, CH, GROUP)

    agg = _sc_scatter(ytab, isrc, idst)
    return agg[0, :N] + agg[1, :N] + h_bias


# speedup vs baseline: 1.0664x; 1.0664x over previous
import functools

import jax
import jax.numpy as jnp
from jax import lax
from jax.experimental import pallas as pl
from jax.experimental.pallas import tpu as pltpu
from jax.experimental.pallas import tpu_sc as plsc

N = 10000
E = 100000
IN = 128
OUT = 128
R = 3
NC = 2
NT = 16
NW = NC * NT
GROUP = 64
NBUF = 4
CH = 8
NCHUNK = 19
NGRP = CH * NCHUNK
EPAD = NW * NGRP * GROUP
ROWS_PAD = 10112
ROWS_PER_TILE = ROWS_PAD // NT

_BM = 1000


def _tc_matmul_body(x_ref, w_ref, y_ref):
    y_ref[...] = jnp.dot(x_ref[...], w_ref[0], preferred_element_type=jnp.float32)


def _tc_matmul(x, weight):
    return pl.pallas_call(
        _tc_matmul_body,
        grid=(R, N // _BM),
        in_specs=[
            pl.BlockSpec((_BM, IN), lambda r, i: (i, 0)),
            pl.BlockSpec((1, IN, OUT), lambda r, i: (r, 0, 0)),
        ],
        out_specs=pl.BlockSpec((_BM, OUT), lambda r, i: (r * (N // _BM) + i, 0)),
        out_shape=jax.ShapeDtypeStruct((R * N, OUT), jnp.float32),
    )(x, weight)


def _sc_scatter_body(ytab_hbm, isrc_hbm, idst_hbm, out_hbm,
                     s0_v, s1_v, d0_v, d1_v, b0, b1, b2, b3, agg_sp,
                     g0, g1, g2, g3, si0, si1, di0, di1):
    c = lax.axis_index("c")
    s = lax.axis_index("s")
    w = c * NT + s
    sbuf = (s0_v, s1_v)
    dbuf = (d0_v, d1_v)
    bufs = (b0, b1, b2, b3)
    gsem = (g0, g1, g2, g3)
    isem = (si0, si1)
    dsem = (di0, di1)

    def load_idx_chunk(k, ring, wait):
        src_hbm = isrc_hbm.at[w, pl.ds(k * CH, CH)]
        dst_hbm = idst_hbm.at[w, pl.ds(k * CH, CH)]
        if wait:
            pltpu.sync_copy(src_hbm, sbuf[ring])
            pltpu.sync_copy(dst_hbm, dbuf[ring])
        else:
            pltpu.async_copy(src_hbm, sbuf[ring], isem[ring])
            pltpu.async_copy(dst_hbm, dbuf[ring], dsem[ring])

    def wait_idx_chunk(ring):
        pltpu.make_async_copy(isrc_hbm.at[w, pl.ds(0, CH)], sbuf[ring], isem[ring]).wait()
        pltpu.make_async_copy(idst_hbm.at[w, pl.ds(0, CH)], dbuf[ring], dsem[ring]).wait()

    def fire_gather(b, ring, grp):
        pltpu.async_copy(ytab_hbm.at[sbuf[ring].at[grp]], bufs[b], gsem[b])

    def wait_gather(b):
        pltpu.make_async_copy(ytab_hbm.at[sbuf[0].at[0]], bufs[b], gsem[b]).wait()

    zval = jnp.zeros((16,), jnp.float32)

    def zero_row(i, carry):
        for j in range(OUT // 16):
            b0[i, pl.ds(j * 16, 16)] = zval
        return carry

    lax.fori_loop(0, GROUP, zero_row, 0)

    load_idx_chunk(0, 0, wait=True)
    for b in range(1, NBUF):
        fire_gather(b, 0, b)
    load_idx_chunk(1, 1, wait=False)
    row0 = s * ROWS_PER_TILE
    for k in range(ROWS_PER_TILE // GROUP):
        pltpu.sync_copy(b0, agg_sp.at[pl.ds(row0 + k * GROUP, GROUP)])
    rem = ROWS_PER_TILE % GROUP
    if rem:
        pltpu.sync_copy(b0.at[pl.ds(0, rem)],
                        agg_sp.at[pl.ds(row0 + (ROWS_PER_TILE // GROUP) * GROUP, rem)])
    fire_gather(0, 0, 0)
    plsc.subcore_barrier()

    def body(k, carry):
        p = lax.rem(k, 2)
        q = lax.rem(k + 1, 2)

        def on_ring(ring_p, ring_q):
            @pl.when(k + 1 < NCHUNK)
            def _():
                wait_idx_chunk(ring_q)
            for b in range(NBUF):
                wait_gather(b)
                pltpu.sync_copy(bufs[b], agg_sp.at[dbuf[ring_p].at[b]], add=True)
                fire_gather(b, ring_p, NBUF + b)
            for b in range(NBUF):
                wait_gather(b)
                pltpu.sync_copy(bufs[b], agg_sp.at[dbuf[ring_p].at[NBUF + b]], add=True)

                @pl.when(k + 1 < NCHUNK)
                def _():
                    fire_gather(b, ring_q, b)

            @pl.when(k + 2 < NCHUNK)
            def _():
                load_idx_chunk(k + 2, ring_p, wait=False)

        @pl.when(p == 0)
        def _():
            on_ring(0, 1)

        @pl.when(p == 1)
        def _():
            on_ring(1, 0)
        return carry

    lax.fori_loop(0, NCHUNK, body, 0)
    plsc.subcore_barrier()
    pltpu.sync_copy(agg_sp.at[pl.ds(row0, ROWS_PER_TILE)],
                    out_hbm.at[c, pl.ds(row0, ROWS_PER_TILE)])


_sc_scatter = functools.partial(
    pl.kernel,
    out_type=jax.ShapeDtypeStruct((NC, ROWS_PAD, OUT), jnp.float32),
    mesh=plsc.VectorSubcoreMesh(core_axis_name="c", subcore_axis_name="s"),
    scratch_types=[
        pltpu.VMEM((CH, GROUP), jnp.int32),
        pltpu.VMEM((CH, GROUP), jnp.int32),
        pltpu.VMEM((CH, GROUP), jnp.int32),
        pltpu.VMEM((CH, GROUP), jnp.int32),
        pltpu.VMEM((GROUP, OUT), jnp.float32),
        pltpu.VMEM((GROUP, OUT), jnp.float32),
        pltpu.VMEM((GROUP, OUT), jnp.float32),
        pltpu.VMEM((GROUP, OUT), jnp.float32),
        pltpu.VMEM_SHARED((ROWS_PAD, OUT), jnp.float32),
        pltpu.SemaphoreType.DMA,
        pltpu.SemaphoreType.DMA,
        pltpu.SemaphoreType.DMA,
        pltpu.SemaphoreType.DMA,
        pltpu.SemaphoreType.DMA,
        pltpu.SemaphoreType.DMA,
        pltpu.SemaphoreType.DMA,
        pltpu.SemaphoreType.DMA,
    ],
)(_sc_scatter_body)


def kernel(x, edge_index_r0, edge_index_r1, edge_index_r2, weight, h_bias):
    ytab = _tc_matmul(x, weight)

    gidx = jnp.concatenate([
        edge_index_r0[0],
        edge_index_r1[0] + N,
        edge_index_r2[0] + 2 * N,
    ])
    dst = jnp.concatenate([edge_index_r0[1], edge_index_r1[1], edge_index_r2[1]])
    pad = EPAD - R * E
    pad_iota = jax.lax.iota(jnp.int32, pad)
    gidx = jnp.concatenate([gidx, pad_iota % (R * N)])
    dst = jnp.concatenate([dst, N + pad_iota % (ROWS_PAD - N)])
    isrc = gidx.reshape(NW, NGRP, GROUP)
    idst = dst.reshape(NW, NGRP, GROUP)

    agg = _sc_scatter(ytab, isrc, idst)
    return agg[0, :N] + agg[1, :N] + h_bias


# speedup vs baseline: 1.1147x; 1.0452x over previous
import functools

import jax
import jax.numpy as jnp
from jax import lax
from jax.experimental import pallas as pl
from jax.experimental.pallas import tpu as pltpu
from jax.experimental.pallas import tpu_sc as plsc

N = 10000
E = 100000
IN = 128
OUT = 128
R = 3
NC = 2
NT = 16
NW = NC * NT
GROUP = 64
NBUF = 4
CH = 8
NCHUNK = 19
NGRP = CH * NCHUNK
EPAD = NW * NGRP * GROUP
ROWS_PAD = 10112
ROWS_PER_TILE = ROWS_PAD // NT

_BM = 2000


def _tc_matmul_body(x_ref, w_ref, y_ref):
    y_ref[...] = jnp.dot(x_ref[...], w_ref[0], preferred_element_type=jnp.float32)


def _tc_matmul(x, weight):
    return pl.pallas_call(
        _tc_matmul_body,
        grid=(R, N // _BM),
        in_specs=[
            pl.BlockSpec((_BM, IN), lambda r, i: (i, 0)),
            pl.BlockSpec((1, IN, OUT), lambda r, i: (r, 0, 0)),
        ],
        out_specs=pl.BlockSpec((_BM, OUT), lambda r, i: (r * (N // _BM) + i, 0)),
        out_shape=jax.ShapeDtypeStruct((R * N, OUT), jnp.float32),
    )(x, weight)


def _sc_scatter_body(ytab_hbm, isrc_hbm, idst_hbm, out_hbm,
                     s0_v, s1_v, d0_v, d1_v, b0, b1, b2, b3, agg_sp,
                     g0, g1, g2, g3, si0, si1, di0, di1):
    c = lax.axis_index("c")
    s = lax.axis_index("s")
    w = c * NT + s
    sbuf = (s0_v, s1_v)
    dbuf = (d0_v, d1_v)
    bufs = (b0, b1, b2, b3)
    gsem = (g0, g1, g2, g3)
    isem = (si0, si1)
    dsem = (di0, di1)

    def load_idx_chunk(k, ring, wait):
        src_hbm = isrc_hbm.at[w, pl.ds(k * CH, CH)]
        dst_hbm = idst_hbm.at[w, pl.ds(k * CH, CH)]
        if wait:
            pltpu.sync_copy(src_hbm, sbuf[ring])
            pltpu.sync_copy(dst_hbm, dbuf[ring])
        else:
            pltpu.async_copy(src_hbm, sbuf[ring], isem[ring])
            pltpu.async_copy(dst_hbm, dbuf[ring], dsem[ring])

    def wait_idx_chunk(ring):
        pltpu.make_async_copy(isrc_hbm.at[w, pl.ds(0, CH)], sbuf[ring], isem[ring]).wait()
        pltpu.make_async_copy(idst_hbm.at[w, pl.ds(0, CH)], dbuf[ring], dsem[ring]).wait()

    def fire_gather(b, ring, grp):
        pltpu.async_copy(ytab_hbm.at[sbuf[ring].at[grp]], bufs[b], gsem[b])

    def wait_gather(b):
        pltpu.make_async_copy(ytab_hbm.at[sbuf[0].at[0]], bufs[b], gsem[b]).wait()

    zval = jnp.zeros((16,), jnp.float32)

    def zero_row(i, carry):
        for j in range(OUT // 16):
            b0[i, pl.ds(j * 16, 16)] = zval
        return carry

    lax.fori_loop(0, GROUP, zero_row, 0)

    load_idx_chunk(0, 0, wait=True)
    for b in range(1, NBUF):
        fire_gather(b, 0, b)
    load_idx_chunk(1, 1, wait=False)
    row0 = s * ROWS_PER_TILE
    for k in range(ROWS_PER_TILE // GROUP):
        pltpu.sync_copy(b0, agg_sp.at[pl.ds(row0 + k * GROUP, GROUP)])
    rem = ROWS_PER_TILE % GROUP
    if rem:
        pltpu.sync_copy(b0.at[pl.ds(0, rem)],
                        agg_sp.at[pl.ds(row0 + (ROWS_PER_TILE // GROUP) * GROUP, rem)])
    fire_gather(0, 0, 0)
    plsc.subcore_barrier()

    def body(k, carry):
        p = lax.rem(k, 2)
        q = lax.rem(k + 1, 2)

        def on_ring(ring_p, ring_q):
            @pl.when(k + 1 < NCHUNK)
            def _():
                wait_idx_chunk(ring_q)
            for b in range(NBUF):
                wait_gather(b)
                pltpu.sync_copy(bufs[b], agg_sp.at[dbuf[ring_p].at[b]], add=True)
                fire_gather(b, ring_p, NBUF + b)
            for b in range(NBUF):
                wait_gather(b)
                pltpu.sync_copy(bufs[b], agg_sp.at[dbuf[ring_p].at[NBUF + b]], add=True)

                @pl.when(k + 1 < NCHUNK)
                def _():
                    fire_gather(b, ring_q, b)

            @pl.when(k + 2 < NCHUNK)
            def _():
                load_idx_chunk(k + 2, ring_p, wait=False)

        @pl.when(p == 0)
        def _():
            on_ring(0, 1)

        @pl.when(p == 1)
        def _():
            on_ring(1, 0)
        return carry

    lax.fori_loop(0, NCHUNK, body, 0)
    plsc.subcore_barrier()
    pltpu.sync_copy(agg_sp.at[pl.ds(row0, ROWS_PER_TILE)],
                    out_hbm.at[c, pl.ds(row0, ROWS_PER_TILE)])


_sc_scatter = functools.partial(
    pl.kernel,
    out_type=jax.ShapeDtypeStruct((NC, ROWS_PAD, OUT), jnp.float32),
    mesh=plsc.VectorSubcoreMesh(core_axis_name="c", subcore_axis_name="s"),
    scratch_types=[
        pltpu.VMEM((CH, GROUP), jnp.int32),
        pltpu.VMEM((CH, GROUP), jnp.int32),
        pltpu.VMEM((CH, GROUP), jnp.int32),
        pltpu.VMEM((CH, GROUP), jnp.int32),
        pltpu.VMEM((GROUP, OUT), jnp.float32),
        pltpu.VMEM((GROUP, OUT), jnp.float32),
        pltpu.VMEM((GROUP, OUT), jnp.float32),
        pltpu.VMEM((GROUP, OUT), jnp.float32),
        pltpu.VMEM_SHARED((ROWS_PAD, OUT), jnp.float32),
        pltpu.SemaphoreType.DMA,
        pltpu.SemaphoreType.DMA,
        pltpu.SemaphoreType.DMA,
        pltpu.SemaphoreType.DMA,
        pltpu.SemaphoreType.DMA,
        pltpu.SemaphoreType.DMA,
        pltpu.SemaphoreType.DMA,
        pltpu.SemaphoreType.DMA,
    ],
)(_sc_scatter_body)


def kernel(x, edge_index_r0, edge_index_r1, edge_index_r2, weight, h_bias):
    ytab = _tc_matmul(x, weight)

    gidx = jnp.concatenate([
        edge_index_r0[0],
        edge_index_r1[0] + N,
        edge_index_r2[0] + 2 * N,
    ])
    dst = jnp.concatenate([edge_index_r0[1], edge_index_r1[1], edge_index_r2[1]])
    pad = EPAD - R * E
    pad_iota = jax.lax.iota(jnp.int32, pad)
    gidx = jnp.concatenate([gidx, pad_iota % (R * N)])
    dst = jnp.concatenate([dst, N + pad_iota % (ROWS_PAD - N)])
    isrc = gidx.reshape(NW, NGRP, GROUP)
    idst = dst.reshape(NW, NGRP, GROUP)

    agg = _sc_scatter(ytab, isrc, idst)
    return agg[0, :N] + agg[1, :N] + h_bias


# speedup vs baseline: 1.1792x; 1.0579x over previous
import functools

import jax
import jax.numpy as jnp
from jax import lax
from jax.experimental import pallas as pl
from jax.experimental.pallas import tpu as pltpu
from jax.experimental.pallas import tpu_sc as plsc

N = 10000
E = 100000
IN = 128
OUT = 128
R = 3
NC = 2
NT = 16
NW = NC * NT
GROUP = 64
NBUF = 4
CH = 8
NCHUNK = 19
NGRP = CH * NCHUNK
EPAD = NW * NGRP * GROUP
ROWS_PAD = 10112
ROWS_PER_TILE = ROWS_PAD // NT

_BM = 2000


def _tc_matmul_body(x_ref, w_ref, y_ref):
    for r in range(R):
        y_ref[r] = jnp.dot(x_ref[...], w_ref[r], preferred_element_type=jnp.float32)


def _tc_matmul(x, weight):
    y = pl.pallas_call(
        _tc_matmul_body,
        grid=(N // _BM,),
        in_specs=[
            pl.BlockSpec((_BM, IN), lambda i: (i, 0)),
            pl.BlockSpec((R, IN, OUT), lambda i: (0, 0, 0)),
        ],
        out_specs=pl.BlockSpec((R, _BM, OUT), lambda i: (0, i, 0)),
        out_shape=jax.ShapeDtypeStruct((R, N, OUT), jnp.float32),
    )(x, weight)
    return y.reshape(R * N, OUT)


def _sc_scatter_body(ytab_hbm, isrc_hbm, idst_hbm, out_hbm,
                     s0_v, s1_v, d0_v, d1_v, b0, b1, b2, b3, agg_sp,
                     g0, g1, g2, g3, si0, si1, di0, di1):
    c = lax.axis_index("c")
    s = lax.axis_index("s")
    w = c * NT + s
    sbuf = (s0_v, s1_v)
    dbuf = (d0_v, d1_v)
    bufs = (b0, b1, b2, b3)
    gsem = (g0, g1, g2, g3)
    isem = (si0, si1)
    dsem = (di0, di1)

    def load_idx_chunk(k, ring, wait):
        src_hbm = isrc_hbm.at[w, pl.ds(k * CH, CH)]
        dst_hbm = idst_hbm.at[w, pl.ds(k * CH, CH)]
        if wait:
            pltpu.sync_copy(src_hbm, sbuf[ring])
            pltpu.sync_copy(dst_hbm, dbuf[ring])
        else:
            pltpu.async_copy(src_hbm, sbuf[ring], isem[ring])
            pltpu.async_copy(dst_hbm, dbuf[ring], dsem[ring])

    def wait_idx_chunk(ring):
        pltpu.make_async_copy(isrc_hbm.at[w, pl.ds(0, CH)], sbuf[ring], isem[ring]).wait()
        pltpu.make_async_copy(idst_hbm.at[w, pl.ds(0, CH)], dbuf[ring], dsem[ring]).wait()

    def fire_gather(b, ring, grp):
        pltpu.async_copy(ytab_hbm.at[sbuf[ring].at[grp]], bufs[b], gsem[b])

    def wait_gather(b):
        pltpu.make_async_copy(ytab_hbm.at[sbuf[0].at[0]], bufs[b], gsem[b]).wait()

    zval = jnp.zeros((16,), jnp.float32)

    def zero_row(i, carry):
        for j in range(OUT // 16):
            b0[i, pl.ds(j * 16, 16)] = zval
        return carry

    lax.fori_loop(0, GROUP, zero_row, 0)

    load_idx_chunk(0, 0, wait=True)
    for b in range(1, NBUF):
        fire_gather(b, 0, b)
    load_idx_chunk(1, 1, wait=False)
    row0 = s * ROWS_PER_TILE
    for k in range(ROWS_PER_TILE // GROUP):
        pltpu.sync_copy(b0, agg_sp.at[pl.ds(row0 + k * GROUP, GROUP)])
    rem = ROWS_PER_TILE % GROUP
    if rem:
        pltpu.sync_copy(b0.at[pl.ds(0, rem)],
                        agg_sp.at[pl.ds(row0 + (ROWS_PER_TILE // GROUP) * GROUP, rem)])
    fire_gather(0, 0, 0)
    plsc.subcore_barrier()

    def body(k, carry):
        p = lax.rem(k, 2)
        q = lax.rem(k + 1, 2)

        def on_ring(ring_p, ring_q):
            @pl.when(k + 1 < NCHUNK)
            def _():
                wait_idx_chunk(ring_q)
            for b in range(NBUF):
                wait_gather(b)
                pltpu.sync_copy(bufs[b], agg_sp.at[dbuf[ring_p].at[b]], add=True)
                fire_gather(b, ring_p, NBUF + b)
            for b in range(NBUF):
                wait_gather(b)
                pltpu.sync_copy(bufs[b], agg_sp.at[dbuf[ring_p].at[NBUF + b]], add=True)

                @pl.when(k + 1 < NCHUNK)
                def _():
                    fire_gather(b, ring_q, b)

            @pl.when(k + 2 < NCHUNK)
            def _():
                load_idx_chunk(k + 2, ring_p, wait=False)

        @pl.when(p == 0)
        def _():
            on_ring(0, 1)

        @pl.when(p == 1)
        def _():
            on_ring(1, 0)
        return carry

    lax.fori_loop(0, NCHUNK, body, 0)
    plsc.subcore_barrier()
    pltpu.sync_copy(agg_sp.at[pl.ds(row0, ROWS_PER_TILE)],
                    out_hbm.at[c, pl.ds(row0, ROWS_PER_TILE)])


_sc_scatter = functools.partial(
    pl.kernel,
    out_type=jax.ShapeDtypeStruct((NC, ROWS_PAD, OUT), jnp.float32),
    mesh=plsc.VectorSubcoreMesh(core_axis_name="c", subcore_axis_name="s"),
    scratch_types=[
        pltpu.VMEM((CH, GROUP), jnp.int32),
        pltpu.VMEM((CH, GROUP), jnp.int32),
        pltpu.VMEM((CH, GROUP), jnp.int32),
        pltpu.VMEM((CH, GROUP), jnp.int32),
        pltpu.VMEM((GROUP, OUT), jnp.float32),
        pltpu.VMEM((GROUP, OUT), jnp.float32),
        pltpu.VMEM((GROUP, OUT), jnp.float32),
        pltpu.VMEM((GROUP, OUT), jnp.float32),
        pltpu.VMEM_SHARED((ROWS_PAD, OUT), jnp.float32),
        pltpu.SemaphoreType.DMA,
        pltpu.SemaphoreType.DMA,
        pltpu.SemaphoreType.DMA,
        pltpu.SemaphoreType.DMA,
        pltpu.SemaphoreType.DMA,
        pltpu.SemaphoreType.DMA,
        pltpu.SemaphoreType.DMA,
        pltpu.SemaphoreType.DMA,
    ],
)(_sc_scatter_body)


def kernel(x, edge_index_r0, edge_index_r1, edge_index_r2, weight, h_bias):
    ytab = _tc_matmul(x, weight)

    gidx = jnp.concatenate([
        edge_index_r0[0],
        edge_index_r1[0] + N,
        edge_index_r2[0] + 2 * N,
    ])
    dst = jnp.concatenate([edge_index_r0[1], edge_index_r1[1], edge_index_r2[1]])
    pad = EPAD - R * E
    pad_iota = jax.lax.iota(jnp.int32, pad)
    gidx = jnp.concatenate([gidx, pad_iota % (R * N)])
    dst = jnp.concatenate([dst, N + pad_iota % (ROWS_PAD - N)])
    isrc = gidx.reshape(NW, NGRP, GROUP)
    idst = dst.reshape(NW, NGRP, GROUP)

    agg = _sc_scatter(ytab, isrc, idst)
    return agg[0, :N] + agg[1, :N] + h_bias
